# Initial kernel scaffold; baseline (speedup 1.0000x reference)
import jax
import jax.numpy as jnp
from jax.experimental import pallas as pl


def kernel(input1, input2):
    raise NotImplementedError("write your pallas kernel here")


# speedup vs baseline: 3.1940x; 3.1940x over previous
import functools

import jax
import jax.numpy as jnp
from jax import lax
from jax.experimental import pallas as pl
from jax.experimental.pallas import tpu as pltpu
from jax.experimental.pallas import tpu_sc as plsc

NC, NS = 2, 16
NW = NC * NS
CH = 128


def _prep_body(H, W, HB, xf_ref, yf_ref, idx_ref, w_ref):
    b = pl.program_id(0)
    hb = pl.program_id(1)
    xf = xf_ref[0]
    yf = yf_ref[0]
    gx = lax.broadcasted_iota(jnp.int32, (HB, W), 1).astype(jnp.float32)
    gy = (lax.broadcasted_iota(jnp.int32, (HB, W), 0)
          + hb * HB).astype(jnp.float32)
    x = jnp.clip(gx + xf, 0.0, float(W - 1))
    y = jnp.clip(gy + yf, 0.0, float(H - 1))
    x0f = jnp.floor(x)
    y0f = jnp.floor(y)
    x0 = x0f.astype(jnp.int32)
    y0 = y0f.astype(jnp.int32)
    x1 = jnp.minimum(x0 + 1, W - 1)
    y1 = jnp.minimum(y0 + 1, H - 1)
    wx = x - x0f
    wy = y - y0f
    base = b * (H * W)
    r0 = base + y0 * W
    r1 = base + y1 * W
    idx_ref[0, 0] = r0 + x0
    idx_ref[1, 0] = r0 + x1
    idx_ref[2, 0] = r1 + x0
    idx_ref[3, 0] = r1 + x1
    u = 1.0 - wx
    v = 1.0 - wy
    w_ref[0, 0] = u * v
    w_ref[1, 0] = wx * v
    w_ref[2, 0] = u * wy
    w_ref[3, 0] = wx * wy


def _prep(xf, yf, H, W, HB):
    B = xf.shape[0]
    grid = (B, H // HB)
    return pl.pallas_call(
        functools.partial(_prep_body, H, W, HB),
        grid=grid,
        in_specs=[
            pl.BlockSpec((1, HB, W), lambda b, h: (b, h, 0)),
            pl.BlockSpec((1, HB, W), lambda b, h: (b, h, 0)),
        ],
        out_specs=[
            pl.BlockSpec((4, 1, HB, W), lambda b, h: (0, b, h, 0)),
            pl.BlockSpec((4, 1, HB, W), lambda b, h: (0, b, h, 0)),
        ],
        out_shape=[
            jax.ShapeDtypeStruct((4, B, H, W), jnp.int32),
            jax.ShapeDtypeStruct((4, B, H, W), jnp.float32),
        ],
    )(xf, yf)


def _sc_gather_interp(table, idx, wgt, P, C):
    PW = P // NW
    nchunk = PW // CH
    mesh = plsc.VectorSubcoreMesh(core_axis_name="c", subcore_axis_name="s")

    @functools.partial(
        pl.kernel,
        out_type=jax.ShapeDtypeStruct((P, C), jnp.float32),
        mesh=mesh,
        scratch_types=[
            pltpu.VMEM((4, CH), jnp.int32),
            pltpu.VMEM((4, CH), jnp.float32),
            pltpu.VMEM((4, CH, C), jnp.float32),
            pltpu.VMEM((CH, C), jnp.float32),
            pltpu.SemaphoreType.DMA,
        ],
        compiler_params=pltpu.CompilerParams(use_tc_tiling_on_sc=False),
    )
    def k(t_hbm, idx_hbm, wgt_hbm, out_hbm, idxb, wv, rows, outv, sem):
        cid = lax.axis_index("c")
        sid = lax.axis_index("s")
        wid = sid * NC + cid
        wbase = wid * PW

        def chunk_body(g, carry):
            base = wbase + g * CH
            pltpu.sync_copy(idx_hbm.at[:, pl.ds(base, CH)], idxb)
            pltpu.sync_copy(wgt_hbm.at[:, pl.ds(base, CH)], wv)
            cps = [
                pltpu.async_copy(t_hbm.at[idxb.at[kk]], rows.at[kk], sem)
                for kk in range(4)
            ]
            for cp in cps:
                cp.wait()

            def grp(v, c2):
                w0v = wv[0, pl.ds(v * 16, 16)]
                w1v = wv[1, pl.ds(v * 16, 16)]
                w2v = wv[2, pl.ds(v * 16, 16)]
                w3v = wv[3, pl.ds(v * 16, 16)]
                for j in range(16):
                    i = v * 16 + j
                    for h in range(C // 16):
                        sl = pl.ds(h * 16, 16)
                        acc = (rows[0, i, sl] * w0v[j]
                               + rows[1, i, sl] * w1v[j]
                               + rows[2, i, sl] * w2v[j]
                               + rows[3, i, sl] * w3v[j])
                        outv[i, sl] = acc
                return c2

            lax.fori_loop(0, CH // 16, grp, 0)
            pltpu.sync_copy(outv, out_hbm.at[pl.ds(base, CH)])
            return carry

        lax.fori_loop(0, nchunk, chunk_body, 0)

    return k(table, idx, wgt)


def kernel(input1, input2):
    input1 = input1.astype(jnp.float32)
    input2 = input2.astype(jnp.float32)
    B, C, H, W = input1.shape
    P = B * H * W
    xf = input2[:, 0]
    yf = input2[:, 1]
    idx, wgt = _prep(xf, yf, H, W, 256)
    table = input1.transpose(0, 2, 3, 1).reshape(P, C)
    out_t = _sc_gather_interp(
        table, idx.reshape(4, P), wgt.reshape(4, P), P, C)
    return out_t.reshape(B, H, W, C).transpose(0, 3, 1, 2)
